# Initial kernel scaffold; baseline (speedup 1.0000x reference)
import jax
import jax.numpy as jnp
from jax.experimental import pallas as pl


def kernel(x, neighbors, Wq, bq, Wk, bk, Wv, bv, relative_bias, Wo, bo):
    raise NotImplementedError("write your pallas kernel here")


# speedup vs baseline: 4.7643x; 4.7643x over previous
import functools
import math

import jax
import jax.numpy as jnp
from jax import lax
from jax.experimental import pallas as pl
from jax.experimental.pallas import tpu as pltpu
from jax.experimental.pallas import tpu_sc as plsc

B, N, C = 4, 2048, 768
NW = 32
RPW = (B * N) // NW
GCH = 64
NCH = RPW // GCH
WPB = N // RPW


def _sc_gather_body(x2d_hbm, nb_hbm, xg_hbm, xg2_hbm,
                    nb_v, idxf_v, idx2f_v, rows_v, sem):
    nc = plsc.get_sparse_core_info().num_cores
    wid = lax.axis_index("s") * nc + lax.axis_index("c")
    base = wid * RPW
    b = base // N
    i0 = base - b * N
    bN = b * N

    pltpu.sync_copy(nb_hbm, nb_v)

    for k in range(RPW // 16):
        c, o = k // (GCH // 16), (k % (GCH // 16)) * 16
        iv = nb_v[pl.ds(i0 + k * 16, 16)]
        i2v = plsc.load_gather(nb_v, [iv])
        idxf_v[c, pl.ds(o, 16)] = iv + bN
        idx2f_v[c, pl.ds(o, 16)] = i2v + bN

    for c in range(NCH):
        pltpu.async_copy(x2d_hbm.at[idxf_v.at[c]], rows_v, sem).wait()
        pltpu.sync_copy(rows_v, xg_hbm.at[pl.ds(base + c * GCH, GCH)])
    for c in range(NCH):
        pltpu.async_copy(x2d_hbm.at[idx2f_v.at[c]], rows_v, sem).wait()
        pltpu.sync_copy(rows_v, xg2_hbm.at[pl.ds(base + c * GCH, GCH)])


@jax.jit
def _sc_gather(x2d, nb1d):
    mesh = plsc.VectorSubcoreMesh(core_axis_name="c", subcore_axis_name="s")
    f = pl.kernel(
        _sc_gather_body,
        out_type=[jax.ShapeDtypeStruct((B * N, C), jnp.float32),
                  jax.ShapeDtypeStruct((B * N, C), jnp.float32)],
        mesh=mesh,
        scratch_types=[
            pltpu.VMEM((N,), jnp.int32),
            pltpu.VMEM((NCH, GCH), jnp.int32),
            pltpu.VMEM((NCH, GCH), jnp.int32),
            pltpu.VMEM((GCH, C), jnp.float32),
            pltpu.SemaphoreType.DMA,
        ],
        compiler_params=pltpu.CompilerParams(needs_layout_passes=False),
    )
    return f(x2d, nb1d)


def _mm_nt(a, w):
    return lax.dot_general(a, w, (((1,), (1,)), ((), ())),
                           preferred_element_type=jnp.float32)


def _wvo_body(wo_ref, wv_ref, bv_ref, wvo_ref, bvo_ref):
    wo = wo_ref[...]
    wvo_ref[...] = lax.dot_general(wo, wv_ref[...], (((1,), (0,)), ((), ())),
                                   preferred_element_type=jnp.float32)
    bvo_ref[...] = _mm_nt(bv_ref[...], wo)


def _colsum_body(xg_ref, out_ref):
    out_ref[...] = jnp.sum(xg_ref[...], axis=0)[None, None, :]


TM = 256


def _main_body(x_ref, xg_ref, xg2_ref, wq_ref, bq_ref, wk_ref, bk_ref,
               wvo_ref, bvo_ref, cs_ref, bo_ref, rb_ref, out_ref):
    i = pl.program_id(0)
    b = i // (N // TM)
    q = _mm_nt(x_ref[...], wq_ref[...]) + bq_ref[...]
    k = _mm_nt(xg_ref[...], wk_ref[...]) + bk_ref[...]
    s = jnp.sum(q * k, axis=1, keepdims=True) + rb_ref[0, 0]
    e = jnp.exp(jnp.minimum(s * (1.0 / math.sqrt(C)), 80.0))
    z = e + (N - 1.0)
    g2o = _mm_nt(xg2_ref[...], wvo_ref[...]) + bvo_ref[...]
    so = _mm_nt(cs_ref[pl.ds(b, 1), :], wvo_ref[...]) + float(N) * bvo_ref[...]
    out_ref[...] = (so + (e - 1.0) * g2o) / z + bo_ref[...]


def _full(shape):
    return pl.BlockSpec(shape, lambda i: (0, 0))


def kernel(x, neighbors, Wq, bq, Wk, bk, Wv, bv, relative_bias, Wo, bo):
    x2d = x.reshape(B * N, C)
    nb1d = neighbors[:, 0]
    xg, xg2 = _sc_gather(x2d, nb1d)

    wvo, bvo = pl.pallas_call(
        _wvo_body,
        out_shape=[jax.ShapeDtypeStruct((C, C), jnp.float32),
                   jax.ShapeDtypeStruct((1, C), jnp.float32)],
    )(Wo, Wv, bv.reshape(1, C))

    cs = pl.pallas_call(
        _colsum_body,
        grid=(B,),
        in_specs=[pl.BlockSpec((N, C), lambda i: (i, 0))],
        out_specs=pl.BlockSpec((1, 1, C), lambda i: (i, 0, 0)),
        out_shape=jax.ShapeDtypeStruct((B, 1, C), jnp.float32),
    )(xg).reshape(B, C)

    row = lambda spec_shape=(TM, C): pl.BlockSpec(spec_shape, lambda i: (i, 0))
    out2d = pl.pallas_call(
        _main_body,
        grid=(B * N // TM,),
        in_specs=[row(), row(), row(),
                  _full((C, C)), _full((1, C)),
                  _full((C, C)), _full((1, C)),
                  _full((C, C)), _full((1, C)),
                  _full((B, C)), _full((1, C)), _full((1, 1))],
        out_specs=row(),
        out_shape=jax.ShapeDtypeStruct((B * N, C), jnp.float32),
        compiler_params=pltpu.CompilerParams(
            dimension_semantics=("arbitrary",)),
    )(x2d, xg, xg2, Wq, bq.reshape(1, C), Wk, bk.reshape(1, C),
      wvo, bvo, cs, bo.reshape(1, C), relative_bias)

    return out2d.reshape(B, N, C)


# speedup vs baseline: 5.6507x; 1.1860x over previous
import math

import jax
import jax.numpy as jnp
from jax import lax
from jax.experimental import pallas as pl
from jax.experimental.pallas import tpu as pltpu
from jax.experimental.pallas import tpu_sc as plsc

B, N, C = 4, 2048, 768
BN = B * N
NW = 32
RPW = BN // NW
GCH = 64
NCH = RPW // GCH
TM = 512
NPB = N // TM
F32 = jnp.float32
BF16 = jnp.bfloat16


def _prep_body(wq_ref, wk_ref, wv_ref, wo_ref, bq_ref, bk_ref,
               bv_ref, rb_ref, m_ref, wvo_ref, u_ref, w_ref,
               bvo_ref, c1_ref):
    wq = wq_ref[...]
    wk = wk_ref[...]
    wo = wo_ref[...]
    m_ref[...] = lax.dot_general(
        wq, wk, (((0,), (0,)), ((), ())),
        preferred_element_type=F32).astype(BF16)
    wvo_ref[...] = lax.dot_general(
        wo, wv_ref[...], (((1,), (0,)), ((), ())),
        preferred_element_type=F32).astype(BF16)
    u_ref[...] = lax.dot_general(
        bk_ref[...], wq, (((1,), (0,)), ((), ())),
        preferred_element_type=F32)
    w_ref[...] = lax.dot_general(
        bq_ref[...], wk, (((1,), (0,)), ((), ())),
        preferred_element_type=F32)
    bvo_ref[...] = lax.dot_general(
        bv_ref[...], wo, (((1,), (1,)), ((), ())),
        preferred_element_type=F32)
    c1_ref[...] = (jnp.sum(bq_ref[...] * bk_ref[...], axis=1,
                           keepdims=True) + rb_ref[...])


def _sc_gather_body(x2d_hbm, nb_hbm, xg_hbm, xg2_hbm,
                    nb_v, idxf_v, idx2f_v, buf0, buf1,
                    semg, sems0, sems1):
    nc = plsc.get_sparse_core_info().num_cores
    wid = lax.axis_index("s") * nc + lax.axis_index("c")
    base = wid * RPW
    b = base // N
    i0 = base - b * N
    bN = b * N

    pltpu.sync_copy(nb_hbm, nb_v)

    for k in range(RPW // 16):
        c, o = k // (GCH // 16), (k % (GCH // 16)) * 16
        iv = nb_v[pl.ds(i0 + k * 16, 16)]
        i2v = plsc.load_gather(nb_v, [iv])
        idxf_v[c, pl.ds(o, 16)] = iv + bN
        idx2f_v[c, pl.ds(o, 16)] = i2v + bN

    chunks = ([(idxf_v.at[c], xg_hbm, c) for c in range(NCH)]
              + [(idx2f_v.at[c], xg2_hbm, c) for c in range(NCH)])
    bufs = (buf0, buf1)
    sems = (sems0, sems1)
    stores = [None] * len(chunks)
    for j, (idx_ref, out_hbm, c) in enumerate(chunks):
        bi = j & 1
        if j >= 2:
            stores[j - 2].wait()
        pltpu.async_copy(x2d_hbm.at[idx_ref], bufs[bi], semg).wait()
        stores[j] = pltpu.async_copy(
            bufs[bi], out_hbm.at[pl.ds(base + c * GCH, GCH)], sems[bi])
    stores[-2].wait()
    stores[-1].wait()


def _sc_gather(x2d, nb1d):
    mesh = plsc.VectorSubcoreMesh(core_axis_name="c", subcore_axis_name="s")
    f = pl.kernel(
        _sc_gather_body,
        out_type=[jax.ShapeDtypeStruct((BN, C), F32),
                  jax.ShapeDtypeStruct((BN, C), F32)],
        mesh=mesh,
        scratch_types=[
            pltpu.VMEM((N,), jnp.int32),
            pltpu.VMEM((NCH, GCH), jnp.int32),
            pltpu.VMEM((NCH, GCH), jnp.int32),
            pltpu.VMEM((GCH, C), F32),
            pltpu.VMEM((GCH, C), F32),
            pltpu.SemaphoreType.DMA,
            pltpu.SemaphoreType.DMA,
            pltpu.SemaphoreType.DMA,
        ],
        compiler_params=pltpu.CompilerParams(needs_layout_passes=False),
    )
    return f(x2d, nb1d)


def _main_body(x_ref, xg_ref, xg2_ref, m_ref, wvo_ref, u_ref, w_ref,
               bvo_ref, c1_ref, bo_ref, out_ref, cs_s, so_s):
    p = pl.program_id(0)
    i = pl.program_id(1)
    b = i // NPB

    @pl.when((p == 0) & (i == 0))
    def _zero():
        cs_s[...] = jnp.zeros_like(cs_s)

    @pl.when(p == 0)
    def _colsum():
        cs_s[pl.ds(b, 1), :] += jnp.sum(xg_ref[...], axis=0, keepdims=True)

    @pl.when((p == 1) & (i == 0))
    def _so():
        so_s[...] = lax.dot_general(
            cs_s[...].astype(BF16), wvo_ref[...], (((1,), (1,)), ((), ())),
            preferred_element_type=F32) + float(N) * bvo_ref[...]

    @pl.when(p == 1)
    def _compute():
        xb = x_ref[...]
        xgb = xg_ref[...]
        pm = lax.dot_general(
            xb.astype(BF16), m_ref[...], (((1,), (0,)), ((), ())),
            preferred_element_type=F32)
        s = (jnp.sum(pm * xgb, axis=1, keepdims=True)
             + jnp.sum(xb * u_ref[...], axis=1, keepdims=True)
             + jnp.sum(xgb * w_ref[...], axis=1, keepdims=True)
             + c1_ref[0, 0])
        e = jnp.exp(jnp.minimum(s * (1.0 / math.sqrt(C)), 80.0))
        z = e + (N - 1.0)
        g2o = lax.dot_general(
            xg2_ref[...].astype(BF16), wvo_ref[...], (((1,), (1,)), ((), ())),
            preferred_element_type=F32) + bvo_ref[...]
        sob = so_s[pl.ds(b, 1), :]
        out_ref[...] = (sob + (e - 1.0) * g2o) / z + bo_ref[...]


def _pin2(shape):
    return pl.BlockSpec(shape, lambda p, i: (0, 0))


def kernel(x, neighbors, Wq, bq, Wk, bk, Wv, bv, relative_bias, Wo, bo):
    x2d = x.reshape(BN, C)
    nb1d = neighbors[:, 0]

    m, wvo, u, w, bvo, c1 = pl.pallas_call(
        _prep_body,
        out_shape=[jax.ShapeDtypeStruct((C, C), BF16),
                   jax.ShapeDtypeStruct((C, C), BF16),
                   jax.ShapeDtypeStruct((1, C), F32),
                   jax.ShapeDtypeStruct((1, C), F32),
                   jax.ShapeDtypeStruct((1, C), F32),
                   jax.ShapeDtypeStruct((1, 1), F32)],
    )(Wq, Wk, Wv, Wo, bq.reshape(1, C), bk.reshape(1, C),
      bv.reshape(1, C), relative_bias)

    xg, xg2 = _sc_gather(x2d, nb1d)

    row_p1 = pl.BlockSpec(
        (TM, C), lambda p, i: (jnp.where(p == 0, 0, i), 0))
    row_both = pl.BlockSpec((TM, C), lambda p, i: (i, 0))
    out2d = pl.pallas_call(
        _main_body,
        grid=(2, BN // TM),
        in_specs=[row_p1, row_both, row_p1,
                  _pin2((C, C)), _pin2((C, C)),
                  _pin2((1, C)), _pin2((1, C)), _pin2((1, C)),
                  _pin2((1, 1)), _pin2((1, C))],
        out_specs=row_p1,
        out_shape=jax.ShapeDtypeStruct((BN, C), F32),
        scratch_shapes=[pltpu.VMEM((B, C), F32), pltpu.VMEM((B, C), F32)],
        compiler_params=pltpu.CompilerParams(
            dimension_semantics=("arbitrary", "arbitrary")),
    )(x2d, xg, xg2, m, wvo, u, w, bvo, c1, bo.reshape(1, C))

    return out2d.reshape(B, N, C)
